# baseline (speedup 1.0000x reference)
import functools

import jax
import jax.numpy as jnp
from jax import lax
from jax.experimental import pallas as pl
from jax.experimental.pallas import tpu as pltpu
from jax.experimental.pallas import tpu_sc as plsc

RES = 512
H = 1080
W = 1920
NPX = H * W
NWORKERS = 32
PX_PER_W = NPX // NWORKERS
C = 1440
NCHUNK = PX_PER_W // C
VPC = C // 16
GROUPS = [C // 2, C // 2]

NTEX = 6 * RES * RES
PLANE = RES * RES
CUBE_WORDS = NTEX * 3
FROWS = 6 * RES
RPS = FROWS // 16
RB = 4
NB = RPS // RB
BT = RB * RES
CSLOT = BT + 8
SRCB = 3 * CSLOT


def _sc_body(cube_hbm, rays_hbm, out_hbm, table_hbm,
             src_v, dst_v, rays_v, i00_v, i10_v,
             wx_v, wy_v, w0_v, w1_v, out_v,
             sem, sem_r, sem_w, sem_ray, sem_o):
    sid = lax.axis_index("s")
    wid = sid * 2 + lax.axis_index("c")
    iota = lax.iota(jnp.int32, 16)
    zz = iota * 0

    def fire_reads(b):
        fr0 = sid * RPS + b * RB
        f = fr0 >> 9
        y0 = fr0 & (RES - 1)
        base = f * (3 * PLANE) + y0 * RES
        so = lax.rem(b, 3) * SRCB
        for ch in range(3):
            pltpu.async_copy(cube_hbm.at[pl.ds(base + ch * PLANE, BT)],
                             src_v.at[pl.ds(so + ch * CSLOT, BT)], sem_r)

    def wait_reads():
        pltpu.make_async_copy(cube_hbm.at[pl.ds(0, 3 * BT)],
                              src_v.at[pl.ds(0, 3 * BT)], sem_r).wait()

    def build(b):
        so = lax.rem(b, 3) * SRCB
        do = (b & 1) * BT

        @plsc.parallel_loop(0, BT // 16, unroll=8)
        def j_body(j):
            rows = do + j * 16 + iota
            for ch in range(3):
                va = src_v[pl.ds(so + ch * CSLOT + j * 16, 16)]
                vb = src_v[pl.ds(so + ch * CSLOT + j * 16 + 1, 16)]
                plsc.store_scatter(dst_v, [rows, zz + ch], va)
                plsc.store_scatter(dst_v, [rows, zz + (4 + ch)], vb)

    def fire_write(b):
        fr0 = sid * RPS + b * RB
        pltpu.async_copy(dst_v.at[pl.ds((b & 1) * BT, BT)],
                         table_hbm.at[pl.ds(fr0 * RES, BT)], sem_w)

    def wait_write(b):
        pltpu.make_async_copy(table_hbm.at[pl.ds(0, BT)],
                              dst_v.at[pl.ds((b & 1) * BT, BT)],
                              sem_w).wait()

    fire_reads(0)
    fire_reads(1)

    def p1_body(b, c):
        @pl.when(b + 2 < NB)
        def _():
            fire_reads(b + 2)

        wait_reads()

        @pl.when(b >= 2)
        def _():
            wait_write(b - 2)

        build(b)
        fire_write(b)
        return c

    lax.fori_loop(0, NB, p1_body, 0)
    wait_write(NB - 2)
    wait_write(NB - 1)
    plsc.subcore_barrier()

    def fire_rays(ci):
        base_px = wid * PX_PER_W + ci * C
        ro = (ci & 1) * (3 * C)
        for p in range(3):
            pltpu.async_copy(rays_hbm.at[pl.ds(p * NPX + base_px, C)],
                             rays_v.at[pl.ds(ro + p * C, C)], sem_ray)

    def wait_rays():
        pltpu.make_async_copy(rays_hbm.at[pl.ds(0, 3 * C)],
                              rays_v.at[pl.ds(0, 3 * C)], sem_ray).wait()

    def compute_chunk(ci, bo):
        ro = (ci & 1) * (3 * C)

        @plsc.parallel_loop(0, VPC, unroll=2)
        def vec_body(i):
            xx = rays_v[pl.ds(ro + i * 16, 16)]
            yy = rays_v[pl.ds(ro + C + i * 16, 16)]
            zz_ = rays_v[pl.ds(ro + 2 * C + i * 16, 16)]
            ax, ay, az = jnp.abs(xx), jnp.abs(yy), jnp.abs(zz_)
            px, py, pz = xx >= 0.0, yy >= 0.0, zz_ >= 0.0
            is_x = (ax >= ay) & (ax >= az)
            is_y = (~is_x) & (ay >= az)
            fb = jnp.where(
                is_x, jnp.where(px, 0, PLANE),
                jnp.where(is_y, jnp.where(py, 2 * PLANE, 3 * PLANE),
                          jnp.where(pz, 4 * PLANE, 5 * PLANE)))
            ma = jnp.maximum(jnp.maximum(jnp.maximum(ax, ay), az), 1e-12)
            sc_ = jnp.where(is_x, jnp.where(px, -zz_, zz_),
                            jnp.where(is_y, xx, jnp.where(pz, xx, -xx)))
            tc_ = jnp.where(is_x, -yy,
                            jnp.where(is_y, jnp.where(py, zz_, -zz_), -yy))
            k = (0.5 * RES) / ma
            fx = sc_ * k + (0.5 * RES - 0.5)
            fy = tc_ * k + (0.5 * RES - 0.5)
            xb = jnp.clip(fx.astype(jnp.int32), 0, RES - 2)
            yb = jnp.clip(fy.astype(jnp.int32), 0, RES - 2)
            wx = jnp.clip(fx, 0.0, RES - 1.0) - xb.astype(jnp.float32)
            wy = jnp.clip(fy, 0.0, RES - 1.0) - yb.astype(jnp.float32)
            s = pl.ds(bo + i * 16, 16)
            i00 = fb | (yb << 9) | xb
            i00_v[s] = i00
            i10_v[s] = i00 + RES
            wx_v[s] = wx
            wy_v[s] = wy

    def fire_gathers(bo):
        off = 0
        for g in GROUPS:
            s = pl.ds(bo + off, g)
            pltpu.async_copy(table_hbm.at[i00_v.at[s]], w0_v.at[s], sem)
            pltpu.async_copy(table_hbm.at[i10_v.at[s]], w1_v.at[s], sem)
            off += g

    def wait_gathers(bo):
        for wv in (w0_v, w1_v):
            pltpu.make_async_copy(table_hbm.at[pl.ds(0, C)],
                                  wv.at[pl.ds(bo, C)], sem).wait()

    def blend_chunk(ci, bo):
        oo = (ci & 1) * (3 * C)

        @plsc.parallel_loop(0, VPC, unroll=2)
        def blend_body(i):
            s = pl.ds(bo + i * 16, 16)
            rows = iota + (bo + i * 16)
            wx = wx_v[s]
            wy = wy_v[s]
            for ch in range(3):
                c00 = plsc.load_gather(w0_v, [rows, zz + ch])
                c01 = plsc.load_gather(w0_v, [rows, zz + (4 + ch)])
                c10 = plsc.load_gather(w1_v, [rows, zz + ch])
                c11 = plsc.load_gather(w1_v, [rows, zz + (4 + ch)])
                top = c00 + wx * (c01 - c00)
                bot = c10 + wx * (c11 - c10)
                o = top + wy * (bot - top)
                out_v[pl.ds(oo + ch * C + i * 16, 16)] = jnp.clip(o, 0.0, 1.0)

    def fire_out(ci):
        base_px = wid * PX_PER_W + ci * C
        oo = (ci & 1) * (3 * C)
        for ch in range(3):
            pltpu.async_copy(out_v.at[pl.ds(oo + ch * C, C)],
                             out_hbm.at[pl.ds(ch * NPX + base_px, C)], sem_o)

    def wait_out():
        pltpu.make_async_copy(rays_hbm.at[pl.ds(0, 3 * C)],
                              out_v.at[pl.ds(0, 3 * C)], sem_o).wait()

    fire_rays(0)
    wait_rays()
    fire_rays(1)
    compute_chunk(0, 0)
    fire_gathers(0)

    def pipe_body(ci, carry):
        bo = (ci & 1) * C

        @pl.when(ci + 1 < NCHUNK)
        def _():
            fire_rays(ci + 1)

        wait_rays()
        compute_chunk(ci, bo)
        wait_gathers(C - bo)
        fire_gathers(bo)

        @pl.when(ci >= 3)
        def _():
            wait_out()

        blend_chunk(ci - 1, C - bo)
        fire_out(ci - 1)
        return carry

    lax.fori_loop(1, NCHUNK, pipe_body, 0)
    wait_gathers(((NCHUNK - 1) & 1) * C)

    @pl.when(NCHUNK >= 3)
    def _():
        wait_out()

    blend_chunk(NCHUNK - 1, ((NCHUNK - 1) & 1) * C)
    fire_out(NCHUNK - 1)
    wait_out()
    wait_out()


@jax.jit
def kernel(rays_d, sky_cube_map):
    cube_flat = jnp.transpose(sky_cube_map, (0, 3, 1, 2)).reshape(CUBE_WORDS)
    rays_flat = jnp.transpose(rays_d, (2, 0, 1)).reshape(3 * NPX)

    sc_fn = functools.partial(
        pl.kernel,
        mesh=plsc.VectorSubcoreMesh(core_axis_name="c", subcore_axis_name="s"),
        compiler_params=pltpu.CompilerParams(needs_layout_passes=False,
                                             use_tc_tiling_on_sc=False),
        out_type=jax.ShapeDtypeStruct((3 * NPX,), jnp.float32),
        scratch_types=[
            pltpu.HBM((NTEX, 8), jnp.float32),
            pltpu.VMEM((3 * SRCB,), jnp.float32),
            pltpu.VMEM((2 * BT, 8), jnp.float32),
            pltpu.VMEM((2 * 3 * C,), jnp.float32),
            pltpu.VMEM((2 * C,), jnp.int32),
            pltpu.VMEM((2 * C,), jnp.int32),
            pltpu.VMEM((2 * C,), jnp.float32),
            pltpu.VMEM((2 * C,), jnp.float32),
            pltpu.VMEM((2 * C, 8), jnp.float32),
            pltpu.VMEM((2 * C, 8), jnp.float32),
            pltpu.VMEM((2 * 3 * C,), jnp.float32),
            pltpu.SemaphoreType.DMA,
            pltpu.SemaphoreType.DMA,
            pltpu.SemaphoreType.DMA,
            pltpu.SemaphoreType.DMA,
            pltpu.SemaphoreType.DMA,
        ],
    )(_sc_body)
    out = sc_fn(cube_flat, rays_flat)
    return out.reshape(3, H, W)


# speedup vs baseline: 1.0026x; 1.0026x over previous
import functools

import jax
import jax.numpy as jnp
from jax import lax
from jax.experimental import pallas as pl
from jax.experimental.pallas import tpu as pltpu
from jax.experimental.pallas import tpu_sc as plsc

RES = 512
H = 1080
W = 1920
NPX = H * W
NWORKERS = 32
PX_PER_W = NPX // NWORKERS
C = 1440
NCHUNK = PX_PER_W // C
VPC = C // 16
GROUPS = [C // 2, C // 2]

NTEX = 6 * RES * RES
PLANE = RES * RES
CUBE_WORDS = NTEX * 3
FROWS = 6 * RES
RPS = FROWS // 16
RB = 4
NB = RPS // RB
BT = RB * RES
CSLOT = BT + 8
SRCB = 3 * CSLOT


def _sc_body(cube_hbm, rays_hbm, out_hbm, table_hbm,
             src_v, dst_v, rays_v, i00_v, i10_v,
             wx_v, wy_v, w0_v, w1_v, out_v,
             sem, sem_r, sem_w, sem_ray, sem_o):
    sid = lax.axis_index("s")
    wid = sid * 2 + lax.axis_index("c")
    iota = lax.iota(jnp.int32, 16)
    zz = iota * 0

    def fire_reads(b):
        fr0 = sid * RPS + b * RB
        f = fr0 >> 9
        y0 = fr0 & (RES - 1)
        base = f * (3 * PLANE) + y0 * RES
        so = lax.rem(b, 3) * SRCB
        for ch in range(3):
            pltpu.async_copy(cube_hbm.at[pl.ds(base + ch * PLANE, BT)],
                             src_v.at[pl.ds(so + ch * CSLOT, BT)], sem_r)

    def wait_reads():
        pltpu.make_async_copy(cube_hbm.at[pl.ds(0, 3 * BT)],
                              src_v.at[pl.ds(0, 3 * BT)], sem_r).wait()

    def build(b):
        so = lax.rem(b, 3) * SRCB
        do = (b & 1) * BT

        @plsc.parallel_loop(0, BT // 16, unroll=4)
        def j_body(j):
            rows = do + j * 16 + iota
            for ch in range(3):
                va = src_v[pl.ds(so + ch * CSLOT + j * 16, 16)]
                vb = src_v[pl.ds(so + ch * CSLOT + j * 16 + 1, 16)]
                plsc.store_scatter(dst_v, [rows, zz + ch], va)
                plsc.store_scatter(dst_v, [rows, zz + (4 + ch)], vb)

    def fire_write(b):
        fr0 = sid * RPS + b * RB
        pltpu.async_copy(dst_v.at[pl.ds((b & 1) * BT, BT)],
                         table_hbm.at[pl.ds(fr0 * RES, BT)], sem_w)

    def wait_write(b):
        pltpu.make_async_copy(table_hbm.at[pl.ds(0, BT)],
                              dst_v.at[pl.ds((b & 1) * BT, BT)],
                              sem_w).wait()

    fire_reads(0)
    fire_reads(1)

    def p1_body(b, c):
        @pl.when(b + 2 < NB)
        def _():
            fire_reads(b + 2)

        wait_reads()

        @pl.when(b >= 2)
        def _():
            wait_write(b - 2)

        build(b)
        fire_write(b)
        return c

    lax.fori_loop(0, NB, p1_body, 0)
    wait_write(NB - 2)
    wait_write(NB - 1)
    plsc.subcore_barrier()

    def fire_rays(ci):
        base_px = wid * PX_PER_W + ci * C
        ro = (ci & 1) * (3 * C)
        for p in range(3):
            pltpu.async_copy(rays_hbm.at[pl.ds(p * NPX + base_px, C)],
                             rays_v.at[pl.ds(ro + p * C, C)], sem_ray)

    def wait_rays():
        pltpu.make_async_copy(rays_hbm.at[pl.ds(0, 3 * C)],
                              rays_v.at[pl.ds(0, 3 * C)], sem_ray).wait()

    def compute_chunk(ci, bo):
        ro = (ci & 1) * (3 * C)

        @plsc.parallel_loop(0, VPC, unroll=2)
        def vec_body(i):
            xx = rays_v[pl.ds(ro + i * 16, 16)]
            yy = rays_v[pl.ds(ro + C + i * 16, 16)]
            zz_ = rays_v[pl.ds(ro + 2 * C + i * 16, 16)]
            ax, ay, az = jnp.abs(xx), jnp.abs(yy), jnp.abs(zz_)
            px, py, pz = xx >= 0.0, yy >= 0.0, zz_ >= 0.0
            is_x = (ax >= ay) & (ax >= az)
            is_y = (~is_x) & (ay >= az)
            face = jnp.where(
                is_x, jnp.where(px, 0, 1),
                jnp.where(is_y, jnp.where(py, 2, 3), jnp.where(pz, 4, 5)))
            ma = jnp.maximum(jnp.maximum(jnp.maximum(ax, ay), az), 1e-12)
            sc_ = jnp.where(is_x, jnp.where(px, -zz_, zz_),
                            jnp.where(is_y, xx, jnp.where(pz, xx, -xx)))
            tc_ = jnp.where(is_x, -yy,
                            jnp.where(is_y, jnp.where(py, zz_, -zz_), -yy))
            k = (0.5 * RES) / ma
            fx = sc_ * k + (0.5 * RES - 0.5)
            fy = tc_ * k + (0.5 * RES - 0.5)
            xb = jnp.clip(fx.astype(jnp.int32), 0, RES - 2)
            yb = jnp.clip(fy.astype(jnp.int32), 0, RES - 2)
            wx = jnp.clip(fx, 0.0, RES - 1.0) - xb.astype(jnp.float32)
            wy = jnp.clip(fy, 0.0, RES - 1.0) - yb.astype(jnp.float32)
            s = pl.ds(bo + i * 16, 16)
            i00 = (face << 18) | (yb << 9) | xb
            i00_v[s] = i00
            i10_v[s] = i00 + RES
            wx_v[s] = wx
            wy_v[s] = wy

    def fire_gathers(bo):
        off = 0
        for g in GROUPS:
            s = pl.ds(bo + off, g)
            pltpu.async_copy(table_hbm.at[i00_v.at[s]], w0_v.at[s], sem)
            pltpu.async_copy(table_hbm.at[i10_v.at[s]], w1_v.at[s], sem)
            off += g

    def wait_gathers(bo):
        for wv in (w0_v, w1_v):
            pltpu.make_async_copy(table_hbm.at[pl.ds(0, C)],
                                  wv.at[pl.ds(bo, C)], sem).wait()

    def blend_chunk(ci, bo):
        oo = (ci & 1) * (3 * C)

        @plsc.parallel_loop(0, VPC, unroll=2)
        def blend_body(i):
            s = pl.ds(bo + i * 16, 16)
            rows = iota + (bo + i * 16)
            wx = wx_v[s]
            wy = wy_v[s]
            for ch in range(3):
                c00 = plsc.load_gather(w0_v, [rows, zz + ch])
                c01 = plsc.load_gather(w0_v, [rows, zz + (4 + ch)])
                c10 = plsc.load_gather(w1_v, [rows, zz + ch])
                c11 = plsc.load_gather(w1_v, [rows, zz + (4 + ch)])
                top = c00 + wx * (c01 - c00)
                bot = c10 + wx * (c11 - c10)
                o = top + wy * (bot - top)
                out_v[pl.ds(oo + ch * C + i * 16, 16)] = jnp.clip(o, 0.0, 1.0)

    def fire_out(ci):
        base_px = wid * PX_PER_W + ci * C
        oo = (ci & 1) * (3 * C)
        for ch in range(3):
            pltpu.async_copy(out_v.at[pl.ds(oo + ch * C, C)],
                             out_hbm.at[pl.ds(ch * NPX + base_px, C)], sem_o)

    def wait_out():
        pltpu.make_async_copy(rays_hbm.at[pl.ds(0, 3 * C)],
                              out_v.at[pl.ds(0, 3 * C)], sem_o).wait()

    fire_rays(0)
    wait_rays()
    fire_rays(1)
    compute_chunk(0, 0)
    fire_gathers(0)

    def pipe_body(ci, carry):
        bo = (ci & 1) * C

        @pl.when(ci + 1 < NCHUNK)
        def _():
            fire_rays(ci + 1)

        wait_rays()
        compute_chunk(ci, bo)
        wait_gathers(C - bo)
        fire_gathers(bo)

        @pl.when(ci >= 3)
        def _():
            wait_out()

        blend_chunk(ci - 1, C - bo)
        fire_out(ci - 1)
        return carry

    lax.fori_loop(1, NCHUNK, pipe_body, 0)
    wait_gathers(((NCHUNK - 1) & 1) * C)

    @pl.when(NCHUNK >= 3)
    def _():
        wait_out()

    blend_chunk(NCHUNK - 1, ((NCHUNK - 1) & 1) * C)
    fire_out(NCHUNK - 1)
    wait_out()
    wait_out()


@jax.jit
def kernel(rays_d, sky_cube_map):
    cube_flat = jnp.transpose(sky_cube_map, (0, 3, 1, 2)).reshape(CUBE_WORDS)
    rays_flat = jnp.transpose(rays_d, (2, 0, 1)).reshape(3 * NPX)

    sc_fn = functools.partial(
        pl.kernel,
        mesh=plsc.VectorSubcoreMesh(core_axis_name="c", subcore_axis_name="s"),
        compiler_params=pltpu.CompilerParams(needs_layout_passes=False,
                                             use_tc_tiling_on_sc=False),
        out_type=jax.ShapeDtypeStruct((3 * NPX,), jnp.float32),
        scratch_types=[
            pltpu.HBM((NTEX, 8), jnp.float32),
            pltpu.VMEM((3 * SRCB,), jnp.float32),
            pltpu.VMEM((2 * BT, 8), jnp.float32),
            pltpu.VMEM((2 * 3 * C,), jnp.float32),
            pltpu.VMEM((2 * C,), jnp.int32),
            pltpu.VMEM((2 * C,), jnp.int32),
            pltpu.VMEM((2 * C,), jnp.float32),
            pltpu.VMEM((2 * C,), jnp.float32),
            pltpu.VMEM((2 * C, 8), jnp.float32),
            pltpu.VMEM((2 * C, 8), jnp.float32),
            pltpu.VMEM((2 * 3 * C,), jnp.float32),
            pltpu.SemaphoreType.DMA,
            pltpu.SemaphoreType.DMA,
            pltpu.SemaphoreType.DMA,
            pltpu.SemaphoreType.DMA,
            pltpu.SemaphoreType.DMA,
        ],
    )(_sc_body)
    out = sc_fn(cube_flat, rays_flat)
    return out.reshape(3, H, W)
